# baseline (speedup 1.0000x reference)
import functools

import jax
import jax.numpy as jnp
from jax import lax
from jax.experimental import pallas as pl
from jax.experimental.pallas import tpu as pltpu
from jax.experimental.pallas import tpu_sc as plsc

NC = 2
NS = 16
NW = NC * NS
CHUNK = 128
GRP = 8
EW = CHUNK // 8
STG = 1024


def _sc_aggregate(n_pad, k_per_worker, d, de):
    rpt = n_pad // NS
    wpt = rpt * de
    xoffs = [o for o in range(0, rpt - CHUNK, CHUNK)] + [rpt - CHUNK]
    eoffs = [o for o in range(0, wpt - STG, STG)] + [wpt - STG]

    mesh = plsc.VectorSubcoreMesh(
        core_axis_name="c", subcore_axis_name="s",
        num_cores=1, num_subcores=NS)

    @functools.partial(
        pl.kernel,
        out_type=[
            jax.ShapeDtypeStruct((n_pad, d), jnp.float32),
            jax.ShapeDtypeStruct((n_pad * de,), jnp.float32),
        ],
        mesh=mesh,
        scratch_types=[
            pltpu.VMEM_SHARED((n_pad, d), jnp.float32),
            pltpu.VMEM_SHARED((n_pad * de,), jnp.float32),
            pltpu.VMEM((GRP, CHUNK), jnp.int32),
            pltpu.VMEM((GRP, CHUNK), jnp.int32),
            pltpu.VMEM((EW, CHUNK), jnp.int32),
            pltpu.VMEM((2, CHUNK, d), jnp.float32),
            pltpu.VMEM((EW, CHUNK), jnp.float32),
            pltpu.VMEM((STG,), jnp.float32),
            pltpu.SemaphoreType.DMA,
            pltpu.SemaphoreType.DMA,
            pltpu.SemaphoreType.DMA,
        ],
    )
    def sc_fn(x_hbm, src_hbm, dst_hbm, ea_hbm, widx_hbm, zx_hbm, ze_hbm,
              accx_hbm, acce_hbm,
              accx_sp, acce_sp, srcbuf, dstbuf, widxbuf, rowbuf, eapay,
              eabuf1, gsem, lsem, ssem):
        s = lax.axis_index("s")
        w = s
        r0 = s * rpt
        e0 = s * wpt

        pltpu.sync_copy(zx_hbm, rowbuf.at[0])
        pltpu.sync_copy(ze_hbm, eabuf1)
        for off in xoffs:
            pltpu.sync_copy(rowbuf.at[0], accx_sp.at[pl.ds(r0 + off, CHUNK)])
        for off in eoffs:
            pltpu.sync_copy(eabuf1, acce_sp.at[pl.ds(e0 + off, STG)])
        plsc.subcore_barrier()

        base = w * k_per_worker

        def stage(g):
            pltpu.sync_copy(src_hbm.at[pl.ds(base + g * GRP, GRP)], srcbuf)
            pltpu.sync_copy(dst_hbm.at[pl.ds(base + g * GRP, GRP)], dstbuf)

        def gather(j2, b):
            pltpu.async_copy(x_hbm.at[srcbuf.at[j2]], rowbuf.at[b], gsem)

        def gather_wait(j2, b):
            pltpu.make_async_copy(
                x_hbm.at[srcbuf.at[j2]], rowbuf.at[b], gsem).wait()

        stage(0)
        gather(0, 0)

        def body(g, carry):
            for j2 in range(GRP):
                b = j2 % 2
                erow = (base + g * GRP + j2) * EW
                lea = pltpu.async_copy(ea_hbm.at[pl.ds(erow, EW)], eapay,
                                      lsem)
                lwx = pltpu.async_copy(widx_hbm.at[pl.ds(erow, EW)],
                                      widxbuf, lsem)
                gather_wait(j2, b)
                if j2 < GRP - 1:
                    gather(j2 + 1, 1 - b)
                lea.wait()
                lwx.wait()
                scs = [pltpu.async_copy(rowbuf.at[b],
                                        accx_sp.at[dstbuf.at[j2]], ssem,
                                        add=True)]
                for r in range(EW):
                    scs.append(pltpu.async_copy(
                        eapay.at[r], acce_sp.at[widxbuf.at[r]], ssem,
                        add=True))
                for desc in scs:
                    desc.wait()
                if j2 == GRP - 1:
                    stage(g + 1)
                    gather(0, 1 - b)
            return carry

        lax.fori_loop(0, k_per_worker // GRP, body, 0)
        gather_wait(0, 0)
        plsc.subcore_barrier()

        for off in xoffs:
            pltpu.sync_copy(accx_sp.at[pl.ds(r0 + off, CHUNK)], rowbuf.at[0])
            pltpu.sync_copy(rowbuf.at[0],
                            accx_hbm.at[pl.ds(r0 + off, CHUNK)])
        for off in eoffs:
            pltpu.sync_copy(acce_sp.at[pl.ds(e0 + off, STG)], eabuf1)
            pltpu.sync_copy(eabuf1, acce_hbm.at[pl.ds(e0 + off, STG)])

    return sc_fn


def _mlp_kernel(x_ref, a0_ref, a1_ref, e0_ref, e1_ref,
                we_ref, w1_ref, b1_ref, w2_ref, b2_ref, out_ref):
    ea = e0_ref[...] + e1_ref[...]
    h = (x_ref[...] + a0_ref[...] + a1_ref[...]
         + jnp.dot(ea, we_ref[...], precision=lax.Precision.HIGHEST,
                   preferred_element_type=jnp.float32))
    t = jnp.maximum(
        jnp.dot(h, w1_ref[...], precision=lax.Precision.HIGHEST,
                preferred_element_type=jnp.float32) + b1_ref[...], 0.0)
    out_ref[...] = (
        jnp.dot(t, w2_ref[...], precision=lax.Precision.HIGHEST,
                preferred_element_type=jnp.float32) + b2_ref[...])


def kernel(x, edge_index, edge_attr, W_e, W1, b1, W2, b2):
    n, d = x.shape
    e = edge_index.shape[1]
    de = edge_attr.shape[1]
    h_dim = W1.shape[1]

    n_pad = ((n + 16 + 127) // 128) * 128
    pad_rows = n_pad - n

    k_per_worker = ((-(-e // (NW * CHUNK)) + 7) // 8) * 8
    e_pad = NW * k_per_worker * CHUNK
    pad_e = e_pad - e

    src = edge_index[0]
    dst = edge_index[1]
    pad_ar = jnp.arange(pad_e, dtype=jnp.int32)
    src_full = jnp.concatenate([src, (pad_ar * 97) % n])
    dst_full = jnp.concatenate([dst, n + (pad_ar % pad_rows)])
    zrows = jnp.zeros((GRP, CHUNK), jnp.int32)
    src_p = src_full.reshape(NW * k_per_worker, CHUNK)
    dst_p = dst_full.reshape(NW * k_per_worker, CHUNK)
    ea_p = jnp.concatenate(
        [edge_attr, jnp.zeros((pad_e, de), jnp.float32)],
        axis=0).reshape(e_pad * de // CHUNK, CHUNK)
    widx = (dst_full[:, None] * de + jnp.arange(de, dtype=jnp.int32)[None, :]
            ).reshape(e_pad * de // CHUNK, CHUNK)

    zx = jnp.zeros((CHUNK, d), jnp.float32)
    ze = jnp.zeros((STG,), jnp.float32)

    sc_fn = _sc_aggregate(n_pad, k_per_worker, d, de)
    rows16 = (NW // 2) * k_per_worker
    halves = []
    for h in (0, 1):
        src_h = jnp.concatenate([src_p[h * rows16:(h + 1) * rows16], zrows])
        dst_h = jnp.concatenate([dst_p[h * rows16:(h + 1) * rows16], zrows])
        ea_h = ea_p[h * rows16 * EW:(h + 1) * rows16 * EW]
        widx_h = widx[h * rows16 * EW:(h + 1) * rows16 * EW]
        halves.append(sc_fn(x, src_h, dst_h, ea_h, widx_h, zx, ze))
    (accx0, acce_f0), (accx1, acce_f1) = halves
    acce0 = acce_f0.reshape(n_pad, de)
    acce1 = acce_f1.reshape(n_pad, de)

    blk = 1000
    grid = n // blk
    out = pl.pallas_call(
        _mlp_kernel,
        grid=(grid,),
        in_specs=[
            pl.BlockSpec((blk, d), lambda i: (i, 0)),
            pl.BlockSpec((blk, d), lambda i: (i, 0)),
            pl.BlockSpec((blk, d), lambda i: (i, 0)),
            pl.BlockSpec((blk, de), lambda i: (i, 0)),
            pl.BlockSpec((blk, de), lambda i: (i, 0)),
            pl.BlockSpec((de, d), lambda i: (0, 0)),
            pl.BlockSpec((d, h_dim), lambda i: (0, 0)),
            pl.BlockSpec((1, h_dim), lambda i: (0, 0)),
            pl.BlockSpec((h_dim, d), lambda i: (0, 0)),
            pl.BlockSpec((1, d), lambda i: (0, 0)),
        ],
        out_specs=pl.BlockSpec((blk, d), lambda i: (i, 0)),
        out_shape=jax.ShapeDtypeStruct((n, d), jnp.float32),
    )(x, accx0[:n], accx1[:n], acce0[:n], acce1[:n],
      W_e, W1, b1.reshape(1, h_dim), W2, b2.reshape(1, d))
    return out


# speedup vs baseline: 1.7057x; 1.7057x over previous
import functools

import jax
import jax.numpy as jnp
from jax import lax
from jax.experimental import pallas as pl
from jax.experimental.pallas import tpu as pltpu
from jax.experimental.pallas import tpu_sc as plsc

NC = 2
NS = 16
NW = NC * NS
CHUNK = 128
GRP = 8
EW = CHUNK // 8
STG = 1024


def _sc_aggregate(n_pad, k_per_worker, d, de, ea_rows):
    rpt = n_pad // NS
    wpt = rpt * de
    xoffs = [o for o in range(0, rpt - CHUNK, CHUNK)] + [rpt - CHUNK]
    eoffs = [o for o in range(0, wpt - STG, STG)] + [wpt - STG]

    mesh = plsc.VectorSubcoreMesh(
        core_axis_name="c", subcore_axis_name="s",
        num_cores=NC, num_subcores=NS)

    @functools.partial(
        pl.kernel,
        out_type=[
            jax.ShapeDtypeStruct((n_pad, d), jnp.float32),
            jax.ShapeDtypeStruct((n_pad, d), jnp.float32),
            jax.ShapeDtypeStruct((n_pad * de,), jnp.float32),
            jax.ShapeDtypeStruct((n_pad * de,), jnp.float32),
        ],
        mesh=mesh,
        scratch_types=[
            pltpu.VMEM_SHARED((n_pad, d), jnp.float32),
            pltpu.VMEM_SHARED((n_pad * de,), jnp.float32),
            pltpu.VMEM((GRP, CHUNK), jnp.int32),
            pltpu.VMEM((GRP, CHUNK), jnp.int32),
            pltpu.VMEM((EW, CHUNK), jnp.int32),
            pltpu.VMEM((2, CHUNK, d), jnp.float32),
            pltpu.VMEM((EW, CHUNK), jnp.float32),
            pltpu.VMEM((STG,), jnp.float32),
            pltpu.SemaphoreType.DMA,
            pltpu.SemaphoreType.DMA,
            pltpu.SemaphoreType.DMA,
        ],
    )
    def sc_fn(x_hbm, src_hbm, dst_hbm, ea_hbm, widx_hbm, zx_hbm, ze_hbm,
              accx0_hbm, accx1_hbm, acce0_hbm, acce1_hbm,
              accx_sp, acce_sp, srcbuf, dstbuf, widxbuf, rowbuf, eapay,
              eabuf1, gsem, lsem, ssem):
        c = lax.axis_index("c")
        s = lax.axis_index("s")
        w = s * NC + c
        r0 = s * rpt
        e0 = s * wpt

        pltpu.sync_copy(zx_hbm, rowbuf.at[0])
        pltpu.sync_copy(ze_hbm, eabuf1)
        for off in xoffs:
            pltpu.sync_copy(rowbuf.at[0], accx_sp.at[pl.ds(r0 + off, CHUNK)])
        for off in eoffs:
            pltpu.sync_copy(eabuf1, acce_sp.at[pl.ds(e0 + off, STG)])
        plsc.subcore_barrier()

        base = w * k_per_worker

        def stage(g):
            pltpu.sync_copy(src_hbm.at[pl.ds(base + g * GRP, GRP)], srcbuf)
            pltpu.sync_copy(dst_hbm.at[pl.ds(base + g * GRP, GRP)], dstbuf)

        def gather(j2, b):
            pltpu.async_copy(x_hbm.at[srcbuf.at[j2]], rowbuf.at[b], gsem)

        def gather_wait(j2, b):
            pltpu.make_async_copy(
                x_hbm.at[srcbuf.at[j2]], rowbuf.at[b], gsem).wait()

        stage(0)
        gather(0, 0)

        def body(g, carry):
            for j2 in range(GRP):
                b = j2 % 2
                erow = (base + g * GRP + j2) * EW
                erow_c = jnp.minimum(erow, ea_rows - EW)
                lea = pltpu.async_copy(ea_hbm.at[pl.ds(erow_c, EW)], eapay,
                                      lsem)
                lwx = pltpu.async_copy(widx_hbm.at[pl.ds(erow, EW)],
                                      widxbuf, lsem)
                gather_wait(j2, b)
                if j2 < GRP - 1:
                    gather(j2 + 1, 1 - b)
                lea.wait()
                lwx.wait()
                scs = [pltpu.async_copy(rowbuf.at[b],
                                        accx_sp.at[dstbuf.at[j2]], ssem,
                                        add=True)]
                for r in range(EW):
                    scs.append(pltpu.async_copy(
                        eapay.at[r], acce_sp.at[widxbuf.at[r]], ssem,
                        add=True))
                for desc in scs:
                    desc.wait()
                if j2 == GRP - 1:
                    stage(g + 1)
                    gather(0, 1 - b)
            return carry

        lax.fori_loop(0, k_per_worker // GRP, body, 0)
        gather_wait(0, 0)
        plsc.subcore_barrier()

        for off in xoffs:
            pltpu.sync_copy(accx_sp.at[pl.ds(r0 + off, CHUNK)], rowbuf.at[0])

            @pl.when(c == 0)
            def _():
                pltpu.sync_copy(rowbuf.at[0],
                                accx0_hbm.at[pl.ds(r0 + off, CHUNK)])

            @pl.when(c == 1)
            def _():
                pltpu.sync_copy(rowbuf.at[0],
                                accx1_hbm.at[pl.ds(r0 + off, CHUNK)])
        for off in eoffs:
            pltpu.sync_copy(acce_sp.at[pl.ds(e0 + off, STG)], eabuf1)

            @pl.when(c == 0)
            def _():
                pltpu.sync_copy(eabuf1, acce0_hbm.at[pl.ds(e0 + off, STG)])

            @pl.when(c == 1)
            def _():
                pltpu.sync_copy(eabuf1, acce1_hbm.at[pl.ds(e0 + off, STG)])

    return sc_fn


def _mlp_kernel(x_ref, a0_ref, a1_ref, e0_ref, e1_ref,
                we_ref, w1_ref, b1_ref, w2_ref, b2_ref, out_ref):
    ea = e0_ref[...] + e1_ref[...]
    h = (x_ref[...] + a0_ref[...] + a1_ref[...]
         + jnp.dot(ea, we_ref[...], precision=lax.Precision.HIGHEST,
                   preferred_element_type=jnp.float32))
    t = jnp.maximum(
        jnp.dot(h, w1_ref[...], precision=lax.Precision.HIGHEST,
                preferred_element_type=jnp.float32) + b1_ref[...], 0.0)
    out_ref[...] = (
        jnp.dot(t, w2_ref[...], precision=lax.Precision.HIGHEST,
                preferred_element_type=jnp.float32) + b2_ref[...])


def kernel(x, edge_index, edge_attr, W_e, W1, b1, W2, b2):
    n, d = x.shape
    e = edge_index.shape[1]
    de = edge_attr.shape[1]
    h_dim = W1.shape[1]

    n_pad = ((n + 16 + 127) // 128) * 128
    pad_rows = n_pad - n

    k_per_worker = ((-(-e // (NW * CHUNK)) + 7) // 8) * 8
    e_pad = NW * k_per_worker * CHUNK
    pad_e = e_pad - e

    src = edge_index[0]
    dst = edge_index[1]
    pad_ar = jnp.arange(pad_e, dtype=jnp.int32)
    src_full = jnp.concatenate([src, (pad_ar * 97) % n])
    dst_full = jnp.concatenate([dst, n + (pad_ar % pad_rows)])
    zrows = jnp.zeros((GRP, CHUNK), jnp.int32)
    src_p = jnp.concatenate(
        [src_full.reshape(NW * k_per_worker, CHUNK), zrows])
    dst_p = jnp.concatenate(
        [dst_full.reshape(NW * k_per_worker, CHUNK), zrows])
    ea_rows = e * de // CHUNK
    ea_p = edge_attr.reshape(ea_rows, CHUNK)
    widx = (dst_full[:, None] * de + jnp.arange(de, dtype=jnp.int32)[None, :]
            ).reshape(e_pad * de // CHUNK, CHUNK)

    zx = jnp.zeros((CHUNK, d), jnp.float32)
    ze = jnp.zeros((STG,), jnp.float32)

    sc_fn = _sc_aggregate(n_pad, k_per_worker, d, de, ea_rows)
    accx0, accx1, acce_f0, acce_f1 = sc_fn(x, src_p, dst_p, ea_p, widx,
                                           zx, ze)
    acce0 = acce_f0.reshape(n_pad, de)
    acce1 = acce_f1.reshape(n_pad, de)

    blk = 1000
    grid = n // blk
    out = pl.pallas_call(
        _mlp_kernel,
        grid=(grid,),
        in_specs=[
            pl.BlockSpec((blk, d), lambda i: (i, 0)),
            pl.BlockSpec((blk, d), lambda i: (i, 0)),
            pl.BlockSpec((blk, d), lambda i: (i, 0)),
            pl.BlockSpec((blk, de), lambda i: (i, 0)),
            pl.BlockSpec((blk, de), lambda i: (i, 0)),
            pl.BlockSpec((de, d), lambda i: (0, 0)),
            pl.BlockSpec((d, h_dim), lambda i: (0, 0)),
            pl.BlockSpec((1, h_dim), lambda i: (0, 0)),
            pl.BlockSpec((h_dim, d), lambda i: (0, 0)),
            pl.BlockSpec((1, d), lambda i: (0, 0)),
        ],
        out_specs=pl.BlockSpec((blk, d), lambda i: (i, 0)),
        out_shape=jax.ShapeDtypeStruct((n, d), jnp.float32),
    )(x, accx0[:n], accx1[:n], acce0[:n], acce1[:n],
      W_e, W1, b1.reshape(1, h_dim), W2, b2.reshape(1, d))
    return out
